# baseline (speedup 1.0000x reference)
import functools

import jax
import jax.numpy as jnp
from jax import lax
from jax.experimental import pallas as pl
from jax.experimental.pallas import tpu as pltpu
from jax.experimental.pallas import tpu_sc as plsc

_B = 16384
_D = 64
_NC = 2
_NS = 16
_NW = _NC * _NS
_RPW = _B // _NW
_CH = 128
_NCH = _RPW // _CH
_GPC = _CH // 16


def _make_sc_kernel():
    mesh = plsc.VectorSubcoreMesh(core_axis_name="c", subcore_axis_name="s")

    @functools.partial(
        pl.kernel,
        out_type=jax.ShapeDtypeStruct((_B,), jnp.float32),
        mesh=mesh,
        scratch_types=[
            pltpu.VMEM((_NCH, _CH), jnp.int32),
            pltpu.VMEM((_NCH, _CH), jnp.int32),
            pltpu.VMEM((_RPW, _D), jnp.float32),
            pltpu.VMEM((_RPW, _D), jnp.float32),
            pltpu.VMEM((_RPW,), jnp.float32),
            pltpu.SemaphoreType.DMA,
            pltpu.SemaphoreType.DMA,
        ],
        compiler_params=pltpu.CompilerParams(
            needs_layout_passes=False, use_tc_tiling_on_sc=False),
    )
    def sc_kernel(ctx_hbm, word_hbm, ctx_tab, word_tab, out_hbm,
                  ctx_idx, word_idx, ctx_rows, word_rows, acc, sem_c, sem_w):
        wid = lax.axis_index("s") * _NC + lax.axis_index("c")
        base = wid * _RPW

        for j in range(_NCH):
            pltpu.sync_copy(ctx_hbm.at[pl.ds(base + j * _CH, _CH)], ctx_idx.at[j])
            pltpu.sync_copy(word_hbm.at[pl.ds(base + j * _CH, _CH)], word_idx.at[j])

        copies = []
        for j in range(_NCH):
            cc = pltpu.async_copy(
                ctx_tab.at[ctx_idx.at[j]], ctx_rows.at[pl.ds(j * _CH, _CH)], sem_c)
            cw = pltpu.async_copy(
                word_tab.at[word_idx.at[j]], word_rows.at[pl.ds(j * _CH, _CH)], sem_w)
            copies.append((cc, cw))

        lane = lax.iota(jnp.int32, 16)

        for j in range(_NCH):
            cc, cw = copies[j]
            cc.wait()
            cw.wait()

            def group_body(g, _, j=j):
                row = lane + (j * _CH + g * 16)
                acc_v = jnp.zeros((16,), jnp.float32)
                for d in range(_D):
                    col = jnp.full((16,), d, jnp.int32)
                    a = plsc.load_gather(ctx_rows, [row, col])
                    b = plsc.load_gather(word_rows, [row, col])
                    acc_v = acc_v + a * b
                acc[pl.ds(j * _CH + g * 16, 16)] = acc_v
                return 0

            lax.fori_loop(0, _GPC, group_body, 0)

        pltpu.sync_copy(acc, out_hbm.at[pl.ds(base, _RPW)])

    return sc_kernel


_sc_kernel = _make_sc_kernel()


def kernel(ctx, word, ctx_table, word_table):
    out = _sc_kernel(ctx.astype(jnp.int32), word.astype(jnp.int32),
                     ctx_table, word_table)
    return out.reshape(_B, 1)


# speedup vs baseline: 1.5518x; 1.5518x over previous
import functools

import jax
import jax.numpy as jnp
from jax import lax
from jax.experimental import pallas as pl
from jax.experimental.pallas import tpu as pltpu
from jax.experimental.pallas import tpu_sc as plsc

_B = 16384
_D = 64
_NC = 2
_NS = 16
_NW = _NC * _NS
_RPW = _B // _NW
_G = 16
_NG = _RPW // _G


def _make_sc_kernel():
    mesh = plsc.VectorSubcoreMesh(core_axis_name="c", subcore_axis_name="s")

    @functools.partial(
        pl.kernel,
        out_type=jax.ShapeDtypeStruct((_B,), jnp.float32),
        mesh=mesh,
        scratch_types=[
            pltpu.VMEM((_RPW,), jnp.int32),
            pltpu.VMEM((_RPW,), jnp.int32),
            pltpu.VMEM((_G, _D), jnp.float32),
            pltpu.VMEM((_G, _D), jnp.float32),
            pltpu.VMEM((_G, _D), jnp.float32),
            pltpu.VMEM((_G, _D), jnp.float32),
            pltpu.VMEM((_RPW,), jnp.float32),
            pltpu.SemaphoreType.DMA,
            pltpu.SemaphoreType.DMA,
            pltpu.SemaphoreType.DMA,
            pltpu.SemaphoreType.DMA,
        ],
        compiler_params=pltpu.CompilerParams(needs_layout_passes=False),
    )
    def sc_kernel(ctx_hbm, word_hbm, ctx_tab, word_tab, out_hbm,
                  ctx_idx, word_idx, rc_a, rc_b, rw_a, rw_b, acc,
                  sem_ca, sem_cb, sem_wa, sem_wb):
        wid = lax.axis_index("s") * _NC + lax.axis_index("c")
        base = wid * _RPW

        pltpu.sync_copy(ctx_hbm.at[pl.ds(base, _RPW)], ctx_idx)
        pltpu.sync_copy(word_hbm.at[pl.ds(base, _RPW)], word_idx)

        lane = lax.iota(jnp.int32, 16)

        def fire(g, rc, rw, sem_c, sem_w):
            icv = ctx_idx[pl.ds(g * _G, _G)]
            iwv = word_idx[pl.ds(g * _G, _G)]
            for r in range(_G):
                pltpu.async_copy(ctx_tab.at[icv[r]], rc.at[r], sem_c)
                pltpu.async_copy(word_tab.at[iwv[r]], rw.at[r], sem_w)

        def drain(rc, rw, sem_c, sem_w):
            for r in range(_G):
                pltpu.make_async_copy(ctx_tab.at[0], rc.at[r], sem_c).wait()
                pltpu.make_async_copy(word_tab.at[0], rw.at[r], sem_w).wait()

        def compute(g, rc, rw):
            acc_v = jnp.zeros((16,), jnp.float32)
            for d in range(_D):
                col = jnp.full((16,), d, jnp.int32)
                a = plsc.load_gather(rc, [lane, col])
                b = plsc.load_gather(rw, [lane, col])
                acc_v = acc_v + a * b
            acc[pl.ds(g * _G, _G)] = acc_v

        fire(0, rc_a, rw_a, sem_ca, sem_wa)

        def body(i, _):
            g0 = 2 * i
            g1 = g0 + 1
            fire(g1, rc_b, rw_b, sem_cb, sem_wb)
            drain(rc_a, rw_a, sem_ca, sem_wa)
            compute(g0, rc_a, rw_a)

            @pl.when(g1 + 1 < _NG)
            def _():
                fire(g1 + 1, rc_a, rw_a, sem_ca, sem_wa)

            drain(rc_b, rw_b, sem_cb, sem_wb)
            compute(g1, rc_b, rw_b)
            return 0

        lax.fori_loop(0, _NG // 2, body, 0)

        pltpu.sync_copy(acc, out_hbm.at[pl.ds(base, _RPW)])

    return sc_kernel


_sc_kernel = _make_sc_kernel()


def kernel(ctx, word, ctx_table, word_table):
    out = _sc_kernel(ctx.astype(jnp.int32), word.astype(jnp.int32),
                     ctx_table, word_table)
    return out.reshape(_B, 1)
